# baseline (speedup 1.0000x reference)
import jax
import jax.numpy as jnp
from jax import lax
from jax.experimental import pallas as pl
from jax.experimental.pallas import tpu as pltpu
from jax.experimental.pallas import tpu_sc as plsc

VOCAB = 1000000
EMB = 64
B_TOTAL = 16384 * 50

NC, NS = 2, 16
NW = NC * NS
B_PER_W = B_TOTAL // NW
CHUNK = 128
NCHUNK = B_PER_W // CHUNK
NBUF = 8
LAG = 2


def _emb_kernel(ids_hbm, table_hbm, out_hbm, idx_v, rows_v, gsems, osems):
    wid = lax.axis_index("c") * NS + lax.axis_index("s")
    base_w = wid * B_PER_W

    pltpu.sync_copy(ids_hbm.at[wid], idx_v)

    def start_gather(g, b):
        pltpu.async_copy(table_hbm.at[idx_v.at[g]], rows_v.at[b], gsems.at[b])

    def wait_gather(g, b):
        pltpu.make_async_copy(table_hbm.at[idx_v.at[g]], rows_v.at[b],
                              gsems.at[b]).wait()

    def out_slot(g):
        return out_hbm.at[pl.ds(base_w + g * CHUNK, CHUNK)]

    def start_store(g, b):
        pltpu.async_copy(rows_v.at[b], out_slot(g), osems.at[b])

    def wait_store(g, b):
        pltpu.make_async_copy(rows_v.at[b], out_slot(g), osems.at[b]).wait()

    for b in range(NBUF - LAG):
        start_gather(b, b)

    for g in range(LAG):
        b = g % NBUF
        wait_gather(g, b)
        start_store(g, b)
        start_gather(g + NBUF - LAG, (g + NBUF - LAG) % NBUF)

    def outer(k):
        for j in range(NBUF):
            g = LAG + k * NBUF + j
            b = (LAG + j) % NBUF
            wait_gather(g, b)
            start_store(g, b)
            wait_store(g - LAG, j)
            start_gather(g + NBUF - LAG, j)

    pl.loop(0, (NCHUNK - NBUF) // NBUF)(outer)

    for g in range(NCHUNK - NBUF + LAG, NCHUNK):
        b = g % NBUF
        wait_gather(g, b)
        start_store(g, b)

    for g in range(NCHUNK - NBUF, NCHUNK):
        wait_store(g, g % NBUF)


def kernel(input_ids, weight):
    ids = input_ids.reshape(NW, NCHUNK, CHUNK).astype(jnp.int32)
    mesh = plsc.VectorSubcoreMesh(core_axis_name="c", subcore_axis_name="s")
    out = pl.kernel(
        _emb_kernel,
        mesh=mesh,
        compiler_params=pltpu.CompilerParams(use_tc_tiling_on_sc=False),
        out_type=jax.ShapeDtypeStruct((B_TOTAL, EMB), jnp.float32),
        scratch_types=[
            pltpu.VMEM((NCHUNK, CHUNK), jnp.int32),
            pltpu.VMEM((NBUF, CHUNK, EMB), jnp.float32),
            pltpu.SemaphoreType.DMA((NBUF,)),
            pltpu.SemaphoreType.DMA((NBUF,)),
        ],
    )(ids, weight)
    return out.reshape(input_ids.shape + (EMB,))
